# baseline (speedup 1.0000x reference)
import functools

import jax
import jax.numpy as jnp
from jax import lax
from jax.experimental import pallas as pl
from jax.experimental.pallas import tpu as pltpu
from jax.experimental.pallas import tpu_sc as plsc

_NC, _NS = 2, 16
_NW = _NC * _NS
_BW = 16384 // _NW


def _g16(vec, idx):
    return vec.at[idx].get(mode="promise_in_bounds")


def _sc_body(x_ref, emb_ref, out_ref, embbuf, xbuf, outbuf, isem, wsem):
    wid = lax.axis_index("s") * _NC + lax.axis_index("c")
    b0 = wid * _BW
    pltpu.sync_copy(emb_ref, embbuf)
    e = embbuf[...]
    lanes = lax.iota(jnp.int32, 16)
    ecol = [_g16(e, (lanes & 3) * 4 + cc) for cc in range(4)]

    def in_slice(lb):
        return x_ref.at[pl.ds(8 * lb, 8), pl.ds(b0, _BW)]

    def out_slice(li):
        return out_ref.at[li, :, pl.ds(b0, _BW)]

    pltpu.async_copy(in_slice(0), xbuf.at[0], isem)

    def band(lb, carry):
        bbuf = lb & 1
        pltpu.make_async_copy(in_slice(lb), xbuf.at[bbuf], isem).wait()

        @pl.when(lb + 1 < 25)
        def _():
            pltpu.async_copy(in_slice(lb + 1), xbuf.at[1 - bbuf], isem)

        for l8 in range(8):
            ob = l8 & 3
            li = 8 * lb + l8

            @pl.when(li >= 4)
            def _():
                pltpu.make_async_copy(outbuf.at[ob], out_slice(li - 4),
                                      wsem).wait()

            def grp(h2, c2):
                for u in range(2):
                    h = 2 * h2 + u
                    xv = xbuf[bbuf, l8, pl.ds(16 * h, 16)]
                    for cc in range(4):
                        vals = _g16(ecol[cc], xv)
                        for k in range(8):
                            outbuf[ob, 4 * k + cc, pl.ds(16 * h, 16)] = vals
                return c2

            lax.fori_loop(0, _BW // 32, grp, 0)
            pltpu.async_copy(outbuf.at[ob], out_slice(li), wsem)
        return carry

    lax.fori_loop(0, 25, band, 0)
    for li in (196, 197, 198, 199):
        pltpu.make_async_copy(outbuf.at[li & 3], out_slice(li), wsem).wait()


def kernel(x, emb_weight):
    B, L = x.shape
    xT = x.T
    emb_flat = emb_weight.reshape(16)
    out = pl.kernel(
        _sc_body,
        out_type=jax.ShapeDtypeStruct((L, 32, B), jnp.float32),
        mesh=plsc.VectorSubcoreMesh(core_axis_name="c", subcore_axis_name="s"),
        compiler_params=pltpu.CompilerParams(use_tc_tiling_on_sc=True),
        scratch_types=[
            pltpu.VMEM((16,), jnp.float32),
            pltpu.VMEM((2, 8, _BW), jnp.int32),
            pltpu.VMEM((4, 32, _BW), jnp.float32),
            pltpu.SemaphoreType.DMA,
            pltpu.SemaphoreType.DMA,
        ],
    )(xT, emb_flat)
    return jnp.transpose(out, (2, 0, 1))


# speedup vs baseline: 1.1142x; 1.1142x over previous
import functools

import jax
import jax.numpy as jnp
from jax import lax
from jax.experimental import pallas as pl
from jax.experimental.pallas import tpu as pltpu
from jax.experimental.pallas import tpu_sc as plsc

_NC, _NS = 2, 16
_NW = _NC * _NS
_BW = 16384 // _NW


def _g16(vec, idx):
    return vec.at[idx].get(mode="promise_in_bounds")


def _sc_body(x_ref, emb_ref, out_ref, embbuf, xbuf, outbuf, isem, wsem):
    wid = lax.axis_index("s") * _NC + lax.axis_index("c")
    b0 = wid * _BW
    pltpu.sync_copy(emb_ref, embbuf)
    e = embbuf[...]
    lanes = lax.iota(jnp.int32, 16)
    ecol = [_g16(e, (lanes & 3) * 4 + cc) for cc in range(4)]

    def in_slice(lb):
        return x_ref.at[pl.ds(8 * lb, 8), pl.ds(b0, _BW)]

    def out_slice(li):
        return out_ref.at[li, :, pl.ds(b0, _BW)]

    pltpu.async_copy(in_slice(0), xbuf.at[0], isem)

    def band(lb, carry):
        bbuf = lb & 1
        pltpu.make_async_copy(in_slice(lb), xbuf.at[bbuf], isem).wait()

        @pl.when(lb + 1 < 25)
        def _():
            pltpu.async_copy(in_slice(lb + 1), xbuf.at[1 - bbuf], isem)

        for l8 in range(8):
            ob = l8 & 1
            li = 8 * lb + l8

            @pl.when(li >= 2)
            def _():
                pltpu.make_async_copy(outbuf.at[ob], out_slice(li - 2),
                                      wsem).wait()

            def grp(h, c2):
                xv = xbuf[bbuf, l8, pl.ds(16 * h, 16)]
                for cc in range(4):
                    vals = _g16(ecol[cc], xv)
                    for k in range(8):
                        outbuf[ob, 4 * k + cc, pl.ds(16 * h, 16)] = vals
                return c2

            lax.fori_loop(0, _BW // 16, grp, 0)
            pltpu.async_copy(outbuf.at[ob], out_slice(li), wsem)
        return carry

    lax.fori_loop(0, 25, band, 0)
    for li in (198, 199):
        pltpu.make_async_copy(outbuf.at[li & 1], out_slice(li), wsem).wait()


def kernel(x, emb_weight):
    B, L = x.shape
    xT = x.T
    emb_flat = emb_weight.reshape(16)
    out = pl.kernel(
        _sc_body,
        out_type=jax.ShapeDtypeStruct((L, 32, B), jnp.float32),
        mesh=plsc.VectorSubcoreMesh(core_axis_name="c", subcore_axis_name="s"),
        compiler_params=pltpu.CompilerParams(use_tc_tiling_on_sc=True),
        scratch_types=[
            pltpu.VMEM((16,), jnp.float32),
            pltpu.VMEM((2, 8, _BW), jnp.int32),
            pltpu.VMEM((2, 32, _BW), jnp.float32),
            pltpu.SemaphoreType.DMA,
            pltpu.SemaphoreType.DMA,
        ],
    )(xT, emb_flat)
    return jnp.transpose(out, (2, 0, 1))


# speedup vs baseline: 1.1689x; 1.0491x over previous
import functools

import jax
import jax.numpy as jnp
from jax import lax
from jax.experimental import pallas as pl
from jax.experimental.pallas import tpu as pltpu
from jax.experimental.pallas import tpu_sc as plsc

_NC, _NS = 2, 16
_NW = _NC * _NS
_BW = 16384 // _NW


def _g16(vec, idx):
    return vec.at[idx].get(mode="promise_in_bounds")


def _sc_body(x_ref, emb_ref, out_ref, embbuf, xbuf, outbuf, isem, wsem):
    wid = lax.axis_index("s") * _NC + lax.axis_index("c")
    b0 = wid * _BW
    pltpu.sync_copy(emb_ref, embbuf)
    e = embbuf[...]
    lanes = lax.iota(jnp.int32, 16)
    ecol = [_g16(e, (lanes & 3) * 4 + cc) for cc in range(4)]

    def in_slice(lb):
        return x_ref.at[pl.ds(8 * lb, 8), pl.ds(b0, _BW)]

    def out_slice(li):
        return out_ref.at[li, :, pl.ds(b0, _BW)]

    pltpu.async_copy(in_slice(0), xbuf.at[0], isem)

    def band(lb, carry):
        bbuf = lb & 1
        pltpu.make_async_copy(in_slice(lb), xbuf.at[bbuf], isem).wait()

        @pl.when(lb + 1 < 25)
        def _():
            pltpu.async_copy(in_slice(lb + 1), xbuf.at[1 - bbuf], isem)

        for l8 in range(8):
            ob = l8 & 1
            li = 8 * lb + l8

            @pl.when(li >= 2)
            def _():
                pltpu.make_async_copy(outbuf.at[ob], out_slice(li - 2),
                                      wsem).wait()

            @functools.partial(plsc.parallel_loop, 0, _BW // 16, unroll=2)
            def _(h):
                xv = xbuf[bbuf, l8, pl.ds(16 * h, 16)]
                for cc in range(4):
                    vals = _g16(ecol[cc], xv)
                    for k in range(8):
                        outbuf[ob, 4 * k + cc, pl.ds(16 * h, 16)] = vals
            pltpu.async_copy(outbuf.at[ob], out_slice(li), wsem)
        return carry

    lax.fori_loop(0, 25, band, 0)
    for li in (198, 199):
        pltpu.make_async_copy(outbuf.at[li & 1], out_slice(li), wsem).wait()


def kernel(x, emb_weight):
    B, L = x.shape
    xT = x.T
    emb_flat = emb_weight.reshape(16)
    out = pl.kernel(
        _sc_body,
        out_type=jax.ShapeDtypeStruct((L, 32, B), jnp.float32),
        mesh=plsc.VectorSubcoreMesh(core_axis_name="c", subcore_axis_name="s"),
        compiler_params=pltpu.CompilerParams(use_tc_tiling_on_sc=True),
        scratch_types=[
            pltpu.VMEM((16,), jnp.float32),
            pltpu.VMEM((2, 8, _BW), jnp.int32),
            pltpu.VMEM((2, 32, _BW), jnp.float32),
            pltpu.SemaphoreType.DMA,
            pltpu.SemaphoreType.DMA,
        ],
    )(xT, emb_flat)
    return jnp.transpose(out, (2, 0, 1))
